# Initial kernel scaffold; baseline (speedup 1.0000x reference)
import jax
import jax.numpy as jnp
from jax.experimental import pallas as pl


def kernel(x, edge_index, edge_weight, edge_index2, edge_weight2, ib1_ln_W, ib1_ln_b, ib1_c1_W, ib1_c1_b, ib1_c2_W, ib1_c2_b, ln1_W, ln1_b, ib2_ln_W, ib2_ln_b, ib2_c1_W, ib2_c1_b, ib2_c2_W, ib2_c2_b, ln2_W, ln2_b, ib3_ln_W, ib3_ln_b, ib3_c1_W, ib3_c1_b, ib3_c2_W, ib3_c2_b, ln3_W, ln3_b):
    raise NotImplementedError("write your pallas kernel here")


# speedup vs baseline: 4.2408x; 4.2408x over previous
import functools

import jax
import jax.numpy as jnp
from jax import lax
from jax.experimental import pallas as pl
from jax.experimental.pallas import tpu as pltpu
from jax.experimental.pallas import tpu_sc as plsc

_NC, _NS = 2, 16
_NW = _NC * _NS


@functools.partial(functools.cache)
def _seg_pair(N, F, E):
    K = 80
    EPW = E // _NW
    NCH = EPW // K
    RPT = (N // _NS) // 8 * 8
    TAIL = N - _NS * RPT
    ZR = 208
    NZ = RPT // ZR
    assert EPW % K == 0 and RPT % ZR == 0 and F % 16 == 0
    assert TAIL % 8 == 0 and TAIL <= ZR

    mesh = plsc.VectorSubcoreMesh(core_axis_name="c", subcore_axis_name="s",
                                  num_cores=_NC, num_subcores=_NS)

    def body(g1, g2, src1, dst1, ew1, src2, dst2, ew2, out,
             table, acc, zbuf, src_v, dst_v, ew_v, rows_v, sem):
        cid = lax.axis_index("c")
        sid = lax.axis_index("s")
        r0 = sid * RPT
        ebase = (cid * _NS + sid) * EPW
        z16 = jnp.zeros((16,), jnp.float32)

        def zrow(i, carry):
            for j in range(F // 16):
                zbuf[i, pl.ds(j * 16, 16)] = z16
            return carry
        lax.fori_loop(0, ZR, zrow, 0)

        def stage(g_hbm):
            pltpu.sync_copy(g_hbm.at[pl.ds(r0, RPT)], table.at[pl.ds(r0, RPT)])
            for i in range(NZ):
                pltpu.sync_copy(zbuf, acc.at[pl.ds(r0 + i * ZR, ZR)])
            if TAIL:
                @pl.when(sid == _NS - 1)
                def _():
                    t0 = _NS * RPT
                    pltpu.sync_copy(g_hbm.at[pl.ds(t0, TAIL)],
                                    table.at[pl.ds(t0, TAIL)])
                    pltpu.sync_copy(zbuf.at[pl.ds(0, TAIL)],
                                    acc.at[pl.ds(t0, TAIL)])

        def edge_loop(src_hbm, dst_hbm, ew_hbm):
            def chunk(i, carry):
                b = ebase + i * K
                pltpu.sync_copy(src_hbm.at[pl.ds(b, K)], src_v)
                pltpu.sync_copy(ew_hbm.at[pl.ds(b, K)], ew_v)
                pltpu.sync_copy(dst_hbm.at[pl.ds(b, K)], dst_v)
                pltpu.async_copy(table.at[src_v], rows_v, sem).wait()
                for kb in range(K // 16):
                    wv = ew_v[pl.ds(kb * 16, 16)]
                    for t in range(16):
                        k = kb * 16 + t
                        w = wv[t]
                        for j in range(F // 16):
                            sl = pl.ds(j * 16, 16)
                            rows_v[k, sl] = rows_v[k, sl] * w
                pltpu.sync_copy(rows_v, acc.at[dst_v], add=True)
                return carry
            lax.fori_loop(0, NCH, chunk, 0)

        def writeback(g):
            pltpu.sync_copy(acc.at[pl.ds(r0, RPT)],
                            out.at[g, cid, pl.ds(r0, RPT)])
            if TAIL:
                @pl.when(sid == _NS - 1)
                def _():
                    t0 = _NS * RPT
                    pltpu.sync_copy(acc.at[pl.ds(t0, TAIL)],
                                    out.at[g, cid, pl.ds(t0, TAIL)])

        stage(g1)
        plsc.subcore_barrier()
        edge_loop(src1, dst1, ew1)
        plsc.subcore_barrier()
        writeback(0)
        stage(g2)
        plsc.subcore_barrier()
        edge_loop(src2, dst2, ew2)
        plsc.subcore_barrier()
        writeback(1)

    return pl.kernel(
        body,
        out_type=jax.ShapeDtypeStruct((2, _NC, N, F), jnp.float32),
        mesh=mesh,
        compiler_params=pltpu.CompilerParams(use_tc_tiling_on_sc=False),
        scratch_types=[
            pltpu.VMEM_SHARED((N, F), jnp.float32),
            pltpu.VMEM_SHARED((N, F), jnp.float32),
            pltpu.VMEM((ZR, F), jnp.float32),
            pltpu.VMEM((K,), jnp.int32),
            pltpu.VMEM((K,), jnp.int32),
            pltpu.VMEM((K,), jnp.float32),
            pltpu.VMEM((K, F), jnp.float32),
            pltpu.SemaphoreType.DMA,
        ],
    )


_BN = 2000


def _first_proj(x, wl, l0, w1, l1, w2, l2):
    N, Din = x.shape
    F = l0.shape[1]

    def body(x_ref, wl_r, l0_r, w1_r, l1_r, w2_r, l2_r, o0, o1, o2):
        xx = x_ref[...]
        o0[...] = jnp.dot(xx, wl_r[...] @ l0_r[...],
                          preferred_element_type=jnp.float32)
        o1[...] = jnp.dot(xx, w1_r[...] @ l1_r[...],
                          preferred_element_type=jnp.float32)
        o2[...] = jnp.dot(xx, w2_r[...] @ l2_r[...],
                          preferred_element_type=jnp.float32)

    full = lambda a: pl.BlockSpec(a.shape, lambda i: (0, 0))
    o = jax.ShapeDtypeStruct((N, F), jnp.float32)
    return pl.pallas_call(
        body,
        grid=(N // _BN,),
        in_specs=[pl.BlockSpec((_BN, Din), lambda i: (i, 0)),
                  full(wl), full(l0), full(w1), full(l1), full(w2), full(l2)],
        out_specs=[pl.BlockSpec((_BN, F), lambda i: (i, 0))] * 3,
        out_shape=[o, o, o],
    )(x, wl, l0, w1, l1, w2, l2)


def _combine_proj(g0, seg, bl, b1, b2, lnb, l0, l1, l2, wl_n, l0_n, w1_n, l1_n,
                  w2_n, l2_n):
    N, F = g0.shape
    Fo = l0_n.shape[1]

    def body(g0_r, seg_r, bl_r, b1_r, b2_r, lnb_r, l0_r, l1_r, l2_r,
             wl_nr, l0_nr, w1_nr, l1_nr, w2_nr, l2_nr, o0, o1, o2):
        c = (bl_r[...] @ l0_r[...] + b1_r[...] @ l1_r[...]
             + b2_r[...] @ l2_r[...] + lnb_r[...])
        s = (g0_r[...] + seg_r[0, 0] + seg_r[0, 1] + seg_r[1, 0]
             + seg_r[1, 1] + c)
        o0[...] = jnp.dot(s, wl_nr[...] @ l0_nr[...],
                          preferred_element_type=jnp.float32)
        o1[...] = jnp.dot(s, w1_nr[...] @ l1_nr[...],
                          preferred_element_type=jnp.float32)
        o2[...] = jnp.dot(s, w2_nr[...] @ l2_nr[...],
                          preferred_element_type=jnp.float32)

    full = lambda a: pl.BlockSpec(a.shape, lambda i: tuple(0 for _ in a.shape))
    o = jax.ShapeDtypeStruct((N, Fo), jnp.float32)
    return pl.pallas_call(
        body,
        grid=(N // _BN,),
        in_specs=[pl.BlockSpec((_BN, F), lambda i: (i, 0)),
                  pl.BlockSpec((2, _NC, _BN, F), lambda i: (0, 0, i, 0)),
                  full(bl), full(b1), full(b2), full(lnb),
                  full(l0), full(l1), full(l2),
                  full(wl_n), full(l0_n), full(w1_n), full(l1_n),
                  full(w2_n), full(l2_n)],
        out_specs=[pl.BlockSpec((_BN, Fo), lambda i: (i, 0))] * 3,
        out_shape=[o, o, o],
    )(g0, seg, bl, b1, b2, lnb, l0, l1, l2, wl_n, l0_n, w1_n, l1_n, w2_n, l2_n)


def _final_stage(g0, seg, bl, b1, b2, lnb, l0, l1, l2, C):
    N, F = g0.shape

    def body(g0_r, seg_r, bl_r, b1_r, b2_r, lnb_r, l0_r, l1_r, l2_r, o):
        c = (bl_r[...] @ l0_r[...] + b1_r[...] @ l1_r[...]
             + b2_r[...] @ l2_r[...] + lnb_r[...])
        s = (g0_r[...] + seg_r[0, 0] + seg_r[0, 1] + seg_r[1, 0]
             + seg_r[1, 1])
        z = s[:, :C] + c
        m = jnp.max(z, axis=1, keepdims=True)
        e = jnp.exp(z - m)
        lse = jnp.log(jnp.sum(e, axis=1, keepdims=True)) + m
        o[...] = z - lse

    full = lambda a: pl.BlockSpec(a.shape, lambda i: tuple(0 for _ in a.shape))
    return pl.pallas_call(
        body,
        grid=(N // _BN,),
        in_specs=[pl.BlockSpec((_BN, F), lambda i: (i, 0)),
                  pl.BlockSpec((2, _NC, _BN, F), lambda i: (0, 0, i, 0)),
                  full(bl), full(b1), full(b2), full(lnb),
                  full(l0), full(l1), full(l2)],
        out_specs=pl.BlockSpec((_BN, C), lambda i: (i, 0)),
        out_shape=jax.ShapeDtypeStruct((N, C), jnp.float32),
    )(g0, seg, bl, b1, b2, lnb, l0, l1, l2)


def kernel(x, edge_index, edge_weight, edge_index2, edge_weight2,
           ib1_ln_W, ib1_ln_b, ib1_c1_W, ib1_c1_b, ib1_c2_W, ib1_c2_b,
           ln1_W, ln1_b,
           ib2_ln_W, ib2_ln_b, ib2_c1_W, ib2_c1_b, ib2_c2_W, ib2_c2_b,
           ln2_W, ln2_b,
           ib3_ln_W, ib3_ln_b, ib3_c1_W, ib3_c1_b, ib3_c2_W, ib3_c2_b,
           ln3_W, ln3_b):
    N, _ = x.shape
    E = edge_weight.shape[0]
    Hd = ib1_ln_W.shape[1]
    C = ln3_W.shape[1]
    Cp = 48

    src1, dst1 = edge_index[0], edge_index[1]
    src2, dst2 = edge_index2[0], edge_index2[1]

    l1 = [ln1_W[i * Hd:(i + 1) * Hd] for i in range(3)]
    l2 = [ln2_W[i * Hd:(i + 1) * Hd] for i in range(3)]
    l3 = [ln3_W[i * C:(i + 1) * C] for i in range(3)]
    l3p = [jnp.pad(m, ((0, 0), (0, Cp - C))) for m in l3]
    row = lambda v: v.reshape(1, -1)

    g0, g1, g2 = _first_proj(x, ib1_ln_W, l1[0], ib1_c1_W, l1[1],
                             ib1_c2_W, l1[2])
    seg = _seg_pair(N, Hd, E)(g1, g2, src1, dst1, edge_weight,
                              src2, dst2, edge_weight2)

    g0, g1, g2 = _combine_proj(g0, seg, row(ib1_ln_b), row(ib1_c1_b),
                               row(ib1_c2_b), row(ln1_b), l1[0], l1[1], l1[2],
                               ib2_ln_W, l2[0], ib2_c1_W, l2[1],
                               ib2_c2_W, l2[2])
    seg = _seg_pair(N, Hd, E)(g1, g2, src1, dst1, edge_weight,
                              src2, dst2, edge_weight2)

    g0, g1, g2 = _combine_proj(g0, seg, row(ib2_ln_b), row(ib2_c1_b),
                               row(ib2_c2_b), row(ln2_b), l2[0], l2[1], l2[2],
                               ib3_ln_W, l3p[0], ib3_c1_W, l3p[1],
                               ib3_c2_W, l3p[2])
    seg = _seg_pair(N, Cp, E)(g1, g2, src1, dst1, edge_weight,
                              src2, dst2, edge_weight2)

    return _final_stage(g0, seg, row(ib3_ln_b), row(ib3_c1_b),
                        row(ib3_c2_b), row(ln3_b), l3[0], l3[1], l3[2], C)


# speedup vs baseline: 11.2774x; 2.6592x over previous
import functools

import jax
import jax.numpy as jnp
from jax import lax
from jax.experimental import pallas as pl
from jax.experimental.pallas import tpu as pltpu
from jax.experimental.pallas import tpu_sc as plsc

_NC, _NS = 2, 16
_NW = _NC * _NS


@functools.partial(functools.cache)
def _seg_pair(N, F, E):
    K = 80
    EPW = E // _NW
    NCH = EPW // K
    RPT = (N // _NS) // 8 * 8
    TAIL = N - _NS * RPT
    ZR = 48
    NZ = RPT // ZR
    assert EPW % K == 0 and RPT % ZR == 0 and F % 16 == 0
    assert TAIL % 8 == 0 and TAIL <= ZR

    mesh = plsc.VectorSubcoreMesh(core_axis_name="c", subcore_axis_name="s",
                                  num_cores=_NC, num_subcores=_NS)

    def body(g1, g2, src1, dst1, ew1, src2, dst2, ew2, out,
             table, acc, zbuf, src_all, dst_all, ew_all, rows_a, rows_b, sem):
        cid = lax.axis_index("c")
        sid = lax.axis_index("s")
        r0 = sid * RPT
        z16 = jnp.zeros((16,), jnp.float32)

        def zrow(i, carry):
            for j in range(F // 16):
                zbuf[i, pl.ds(j * 16, 16)] = z16
            return carry
        lax.fori_loop(0, ZR, zrow, 0)

        def stage(g_hbm):
            pltpu.sync_copy(g_hbm.at[pl.ds(r0, RPT)], table.at[pl.ds(r0, RPT)])
            for i in range(NZ):
                pltpu.sync_copy(zbuf, acc.at[pl.ds(r0 + i * ZR, ZR)])
            if TAIL:
                @pl.when(sid == _NS - 1)
                def _():
                    t0 = _NS * RPT
                    pltpu.sync_copy(g_hbm.at[pl.ds(t0, TAIL)],
                                    table.at[pl.ds(t0, TAIL)])
                    pltpu.sync_copy(zbuf.at[pl.ds(0, TAIL)],
                                    acc.at[pl.ds(t0, TAIL)])

        def multiply(rows, ci):
            for kb in range(K // 16):
                wv = ew_all[ci, pl.ds(kb * 16, 16)]
                for t in range(16):
                    k = kb * 16 + t
                    w = wv[t]
                    for j in range(F // 16):
                        sl = pl.ds(j * 16, 16)
                        rows[k, sl] = rows[k, sl] * w

        def edge_loop(src_hbm, dst_hbm, ew_hbm):
            wid = cid * _NS + sid
            pltpu.sync_copy(src_hbm.at[wid], src_all)
            pltpu.sync_copy(dst_hbm.at[wid], dst_all)
            pltpu.sync_copy(ew_hbm.at[wid], ew_all)

            def pair(p, carry):
                i0 = 2 * p
                dA = pltpu.async_copy(table.at[src_all.at[i0]], rows_a, sem)

                @pl.when(p > 0)
                def _():
                    multiply(rows_b, i0 - 1)
                    pltpu.sync_copy(rows_b, acc.at[dst_all.at[i0 - 1]],
                                    add=True)
                dA.wait()

                @pl.when(i0 + 1 < NCH)
                def _():
                    dB = pltpu.async_copy(table.at[src_all.at[i0 + 1]],
                                          rows_b, sem)
                    multiply(rows_a, i0)
                    pltpu.sync_copy(rows_a, acc.at[dst_all.at[i0]], add=True)
                    dB.wait()

                @pl.when(i0 + 1 >= NCH)
                def _():
                    multiply(rows_a, i0)
                    pltpu.sync_copy(rows_a, acc.at[dst_all.at[i0]], add=True)
                return carry
            lax.fori_loop(0, (NCH + 1) // 2, pair, 0)

        def writeback(g):
            pltpu.sync_copy(acc.at[pl.ds(r0, RPT)],
                            out.at[g, cid, pl.ds(r0, RPT)])
            if TAIL:
                @pl.when(sid == _NS - 1)
                def _():
                    t0 = _NS * RPT
                    pltpu.sync_copy(acc.at[pl.ds(t0, TAIL)],
                                    out.at[g, cid, pl.ds(t0, TAIL)])

        stage(g1)
        plsc.subcore_barrier()
        edge_loop(src1, dst1, ew1)
        plsc.subcore_barrier()
        writeback(0)
        stage(g2)
        plsc.subcore_barrier()
        edge_loop(src2, dst2, ew2)
        plsc.subcore_barrier()
        writeback(1)

    return pl.kernel(
        body,
        out_type=jax.ShapeDtypeStruct((2, _NC, N, F), jnp.float32),
        mesh=mesh,
        compiler_params=pltpu.CompilerParams(use_tc_tiling_on_sc=False),
        scratch_types=[
            pltpu.VMEM_SHARED((N, F), jnp.float32),
            pltpu.VMEM_SHARED((N, F), jnp.float32),
            pltpu.VMEM((ZR, F), jnp.float32),
            pltpu.VMEM((NCH, K), jnp.int32),
            pltpu.VMEM((NCH, K), jnp.int32),
            pltpu.VMEM((NCH, K), jnp.float32),
            pltpu.VMEM((K, F), jnp.float32),
            pltpu.VMEM((K, F), jnp.float32),
            pltpu.SemaphoreType.DMA,
        ],
    )


_BN = 2000


def _first_proj(x, wl, l0, w1, l1, w2, l2):
    N, Din = x.shape
    F = l0.shape[1]

    def body(x_ref, wl_r, l0_r, w1_r, l1_r, w2_r, l2_r, o0, o1, o2):
        xx = x_ref[...]
        o0[...] = jnp.dot(xx, wl_r[...] @ l0_r[...],
                          preferred_element_type=jnp.float32)
        o1[...] = jnp.dot(xx, w1_r[...] @ l1_r[...],
                          preferred_element_type=jnp.float32)
        o2[...] = jnp.dot(xx, w2_r[...] @ l2_r[...],
                          preferred_element_type=jnp.float32)

    full = lambda a: pl.BlockSpec(a.shape, lambda i: (0, 0))
    o = jax.ShapeDtypeStruct((N, F), jnp.float32)
    return pl.pallas_call(
        body,
        grid=(N // _BN,),
        in_specs=[pl.BlockSpec((_BN, Din), lambda i: (i, 0)),
                  full(wl), full(l0), full(w1), full(l1), full(w2), full(l2)],
        out_specs=[pl.BlockSpec((_BN, F), lambda i: (i, 0))] * 3,
        out_shape=[o, o, o],
    )(x, wl, l0, w1, l1, w2, l2)


def _combine_proj(g0, seg, bl, b1, b2, lnb, l0, l1, l2, wl_n, l0_n, w1_n, l1_n,
                  w2_n, l2_n):
    N, F = g0.shape
    Fo = l0_n.shape[1]

    def body(g0_r, seg_r, bl_r, b1_r, b2_r, lnb_r, l0_r, l1_r, l2_r,
             wl_nr, l0_nr, w1_nr, l1_nr, w2_nr, l2_nr, o0, o1, o2):
        c = (bl_r[...] @ l0_r[...] + b1_r[...] @ l1_r[...]
             + b2_r[...] @ l2_r[...] + lnb_r[...])
        s = (g0_r[...] + seg_r[0, 0] + seg_r[0, 1] + seg_r[1, 0]
             + seg_r[1, 1] + c)
        o0[...] = jnp.dot(s, wl_nr[...] @ l0_nr[...],
                          preferred_element_type=jnp.float32)
        o1[...] = jnp.dot(s, w1_nr[...] @ l1_nr[...],
                          preferred_element_type=jnp.float32)
        o2[...] = jnp.dot(s, w2_nr[...] @ l2_nr[...],
                          preferred_element_type=jnp.float32)

    full = lambda a: pl.BlockSpec(a.shape, lambda i: tuple(0 for _ in a.shape))
    o = jax.ShapeDtypeStruct((N, Fo), jnp.float32)
    return pl.pallas_call(
        body,
        grid=(N // _BN,),
        in_specs=[pl.BlockSpec((_BN, F), lambda i: (i, 0)),
                  pl.BlockSpec((2, _NC, _BN, F), lambda i: (0, 0, i, 0)),
                  full(bl), full(b1), full(b2), full(lnb),
                  full(l0), full(l1), full(l2),
                  full(wl_n), full(l0_n), full(w1_n), full(l1_n),
                  full(w2_n), full(l2_n)],
        out_specs=[pl.BlockSpec((_BN, Fo), lambda i: (i, 0))] * 3,
        out_shape=[o, o, o],
    )(g0, seg, bl, b1, b2, lnb, l0, l1, l2, wl_n, l0_n, w1_n, l1_n, w2_n, l2_n)


def _final_stage(g0, seg, bl, b1, b2, lnb, l0, l1, l2, C):
    N, F = g0.shape

    def body(g0_r, seg_r, bl_r, b1_r, b2_r, lnb_r, l0_r, l1_r, l2_r, o):
        c = (bl_r[...] @ l0_r[...] + b1_r[...] @ l1_r[...]
             + b2_r[...] @ l2_r[...] + lnb_r[...])
        s = (g0_r[...] + seg_r[0, 0] + seg_r[0, 1] + seg_r[1, 0]
             + seg_r[1, 1])
        z = s[:, :C] + c
        m = jnp.max(z, axis=1, keepdims=True)
        e = jnp.exp(z - m)
        lse = jnp.log(jnp.sum(e, axis=1, keepdims=True)) + m
        o[...] = z - lse

    full = lambda a: pl.BlockSpec(a.shape, lambda i: tuple(0 for _ in a.shape))
    return pl.pallas_call(
        body,
        grid=(N // _BN,),
        in_specs=[pl.BlockSpec((_BN, F), lambda i: (i, 0)),
                  pl.BlockSpec((2, _NC, _BN, F), lambda i: (0, 0, i, 0)),
                  full(bl), full(b1), full(b2), full(lnb),
                  full(l0), full(l1), full(l2)],
        out_specs=pl.BlockSpec((_BN, C), lambda i: (i, 0)),
        out_shape=jax.ShapeDtypeStruct((N, C), jnp.float32),
    )(g0, seg, bl, b1, b2, lnb, l0, l1, l2)


def kernel(x, edge_index, edge_weight, edge_index2, edge_weight2,
           ib1_ln_W, ib1_ln_b, ib1_c1_W, ib1_c1_b, ib1_c2_W, ib1_c2_b,
           ln1_W, ln1_b,
           ib2_ln_W, ib2_ln_b, ib2_c1_W, ib2_c1_b, ib2_c2_W, ib2_c2_b,
           ln2_W, ln2_b,
           ib3_ln_W, ib3_ln_b, ib3_c1_W, ib3_c1_b, ib3_c2_W, ib3_c2_b,
           ln3_W, ln3_b):
    N, _ = x.shape
    E = edge_weight.shape[0]
    Hd = ib1_ln_W.shape[1]
    C = ln3_W.shape[1]
    Cp = 48

    K = 80
    NCH = E // _NW // K
    esh = lambda v: v.reshape(_NW, NCH, K)
    src1, dst1 = esh(edge_index[0]), esh(edge_index[1])
    src2, dst2 = esh(edge_index2[0]), esh(edge_index2[1])
    ew1, ew2 = esh(edge_weight), esh(edge_weight2)

    l1 = [ln1_W[i * Hd:(i + 1) * Hd] for i in range(3)]
    l2 = [ln2_W[i * Hd:(i + 1) * Hd] for i in range(3)]
    l3 = [ln3_W[i * C:(i + 1) * C] for i in range(3)]
    l3p = [jnp.pad(m, ((0, 0), (0, Cp - C))) for m in l3]
    row = lambda v: v.reshape(1, -1)

    g0, g1, g2 = _first_proj(x, ib1_ln_W, l1[0], ib1_c1_W, l1[1],
                             ib1_c2_W, l1[2])
    seg = _seg_pair(N, Hd, E)(g1, g2, src1, dst1, ew1, src2, dst2, ew2)

    g0, g1, g2 = _combine_proj(g0, seg, row(ib1_ln_b), row(ib1_c1_b),
                               row(ib1_c2_b), row(ln1_b), l1[0], l1[1], l1[2],
                               ib2_ln_W, l2[0], ib2_c1_W, l2[1],
                               ib2_c2_W, l2[2])
    seg = _seg_pair(N, Hd, E)(g1, g2, src1, dst1, ew1, src2, dst2, ew2)

    g0, g1, g2 = _combine_proj(g0, seg, row(ib2_ln_b), row(ib2_c1_b),
                               row(ib2_c2_b), row(ln2_b), l2[0], l2[1], l2[2],
                               ib3_ln_W, l3p[0], ib3_c1_W, l3p[1],
                               ib3_c2_W, l3p[2])
    seg = _seg_pair(N, Cp, E)(g1, g2, src1, dst1, ew1, src2, dst2, ew2)

    return _final_stage(g0, seg, row(ib3_ln_b), row(ib3_c1_b),
                        row(ib3_c2_b), row(ln3_b), l3[0], l3[1], l3[2], C)


# speedup vs baseline: 11.8862x; 1.0540x over previous
import functools

import jax
import jax.numpy as jnp
from jax import lax
from jax.experimental import pallas as pl
from jax.experimental.pallas import tpu as pltpu
from jax.experimental.pallas import tpu_sc as plsc

_NC, _NS = 2, 16
_NW = _NC * _NS


@functools.partial(functools.cache)
def _seg_pair(N, F, E):
    K = 80
    EPW = E // _NW
    NCH = EPW // K
    RPT = (N // _NS) // 8 * 8
    TAIL = N - _NS * RPT
    ZR = 48
    NZ = RPT // ZR
    assert EPW % K == 0 and RPT % ZR == 0 and F % 16 == 0
    assert TAIL % 8 == 0 and TAIL <= ZR

    mesh = plsc.VectorSubcoreMesh(core_axis_name="c", subcore_axis_name="s",
                                  num_cores=_NC, num_subcores=_NS)

    def body(g1, g2, src1, dst1, ew1, src2, dst2, ew2, out,
             table, acc, zbuf, src_all, dst_all, ew_all,
             rows_a, rows_b, rows_c, gsem0, gsem1, gsem2,
             ssem0, ssem1, ssem2):
        cid = lax.axis_index("c")
        sid = lax.axis_index("s")
        r0 = sid * RPT
        z16 = jnp.zeros((16,), jnp.float32)

        def zrow(i, carry):
            for j in range(F // 16):
                zbuf[i, pl.ds(j * 16, 16)] = z16
            return carry
        lax.fori_loop(0, ZR, zrow, 0)

        def stage(g_hbm):
            pltpu.sync_copy(g_hbm.at[pl.ds(r0, RPT)], table.at[pl.ds(r0, RPT)])
            for i in range(NZ):
                pltpu.sync_copy(zbuf, acc.at[pl.ds(r0 + i * ZR, ZR)])
            if TAIL:
                @pl.when(sid == _NS - 1)
                def _():
                    t0 = _NS * RPT
                    pltpu.sync_copy(g_hbm.at[pl.ds(t0, TAIL)],
                                    table.at[pl.ds(t0, TAIL)])
                    pltpu.sync_copy(zbuf.at[pl.ds(0, TAIL)],
                                    acc.at[pl.ds(t0, TAIL)])

        def multiply(rows, ci):
            for kb in range(K // 16):
                wv = ew_all[ci, pl.ds(kb * 16, 16)]
                for t in range(16):
                    k = kb * 16 + t
                    w = wv[t]
                    for j in range(F // 16):
                        sl = pl.ds(j * 16, 16)
                        rows[k, sl] = rows[k, sl] * w

        def edge_loop(src_hbm, dst_hbm, ew_hbm):
            wid = cid * _NS + sid
            pltpu.sync_copy(src_hbm.at[wid], src_all)
            pltpu.sync_copy(dst_hbm.at[wid], dst_all)
            pltpu.sync_copy(ew_hbm.at[wid], ew_all)

            rows = [rows_a, rows_b, rows_c]
            gsem = [gsem0, gsem1, gsem2]
            ssem = [ssem0, ssem1, ssem2]

            def handle(i, u):
                @pl.when((i >= 3) & (i < NCH))
                def _():
                    pltpu.make_async_copy(
                        rows[u], acc.at[dst_all.at[i - 3]], ssem[u],
                    ).wait()

                @pl.when(i < NCH)
                def _():
                    pltpu.async_copy(table.at[src_all.at[i]], rows[u],
                                     gsem[u])

                v = (u - 1) % 3

                @pl.when((i >= 1) & (i - 1 < NCH))
                def _():
                    pltpu.make_async_copy(table.at[src_all.at[i - 1]],
                                          rows[v], gsem[v]).wait()
                    multiply(rows[v], i - 1)
                    pltpu.async_copy(rows[v], acc.at[dst_all.at[i - 1]],
                                     ssem[v], add=True)

            NIT = (NCH + 1 + 2) // 3

            def step(t, carry):
                for u in range(3):
                    handle(3 * t + u, u)
                return carry
            lax.fori_loop(0, NIT, step, 0)

            for c in range(max(0, NCH - 3), NCH):
                pltpu.make_async_copy(rows[c % 3], acc.at[dst_all.at[c]],
                                      ssem[c % 3]).wait()

        def writeback(g):
            pltpu.sync_copy(acc.at[pl.ds(r0, RPT)],
                            out.at[g, cid, pl.ds(r0, RPT)])
            if TAIL:
                @pl.when(sid == _NS - 1)
                def _():
                    t0 = _NS * RPT
                    pltpu.sync_copy(acc.at[pl.ds(t0, TAIL)],
                                    out.at[g, cid, pl.ds(t0, TAIL)])

        stage(g1)
        plsc.subcore_barrier()
        edge_loop(src1, dst1, ew1)
        plsc.subcore_barrier()
        writeback(0)
        stage(g2)
        plsc.subcore_barrier()
        edge_loop(src2, dst2, ew2)
        plsc.subcore_barrier()
        writeback(1)

    return pl.kernel(
        body,
        out_type=jax.ShapeDtypeStruct((2, _NC, N, F), jnp.float32),
        mesh=mesh,
        compiler_params=pltpu.CompilerParams(use_tc_tiling_on_sc=False),
        scratch_types=[
            pltpu.VMEM_SHARED((N, F), jnp.float32),
            pltpu.VMEM_SHARED((N, F), jnp.float32),
            pltpu.VMEM((ZR, F), jnp.float32),
            pltpu.VMEM((NCH, K), jnp.int32),
            pltpu.VMEM((NCH, K), jnp.int32),
            pltpu.VMEM((NCH, K), jnp.float32),
            pltpu.VMEM((K, F), jnp.float32),
            pltpu.VMEM((K, F), jnp.float32),
            pltpu.VMEM((K, F), jnp.float32),
            pltpu.SemaphoreType.DMA,
            pltpu.SemaphoreType.DMA,
            pltpu.SemaphoreType.DMA,
            pltpu.SemaphoreType.DMA,
            pltpu.SemaphoreType.DMA,
            pltpu.SemaphoreType.DMA,
        ],
    )


_BN = 2000


def _first_proj(x, wl, l0, w1, l1, w2, l2):
    N, Din = x.shape
    F = l0.shape[1]

    def body(x_ref, wl_r, l0_r, w1_r, l1_r, w2_r, l2_r, o0, o1, o2):
        xx = x_ref[...]
        o0[...] = jnp.dot(xx, wl_r[...] @ l0_r[...],
                          preferred_element_type=jnp.float32)
        o1[...] = jnp.dot(xx, w1_r[...] @ l1_r[...],
                          preferred_element_type=jnp.float32)
        o2[...] = jnp.dot(xx, w2_r[...] @ l2_r[...],
                          preferred_element_type=jnp.float32)

    full = lambda a: pl.BlockSpec(a.shape, lambda i: (0, 0))
    o = jax.ShapeDtypeStruct((N, F), jnp.float32)
    return pl.pallas_call(
        body,
        grid=(N // _BN,),
        in_specs=[pl.BlockSpec((_BN, Din), lambda i: (i, 0)),
                  full(wl), full(l0), full(w1), full(l1), full(w2), full(l2)],
        out_specs=[pl.BlockSpec((_BN, F), lambda i: (i, 0))] * 3,
        out_shape=[o, o, o],
    )(x, wl, l0, w1, l1, w2, l2)


def _combine_proj(g0, seg, bl, b1, b2, lnb, l0, l1, l2, wl_n, l0_n, w1_n, l1_n,
                  w2_n, l2_n):
    N, F = g0.shape
    Fo = l0_n.shape[1]

    def body(g0_r, seg_r, bl_r, b1_r, b2_r, lnb_r, l0_r, l1_r, l2_r,
             wl_nr, l0_nr, w1_nr, l1_nr, w2_nr, l2_nr, o0, o1, o2):
        c = (bl_r[...] @ l0_r[...] + b1_r[...] @ l1_r[...]
             + b2_r[...] @ l2_r[...] + lnb_r[...])
        s = (g0_r[...] + seg_r[0, 0] + seg_r[0, 1] + seg_r[1, 0]
             + seg_r[1, 1] + c)
        o0[...] = jnp.dot(s, wl_nr[...] @ l0_nr[...],
                          preferred_element_type=jnp.float32)
        o1[...] = jnp.dot(s, w1_nr[...] @ l1_nr[...],
                          preferred_element_type=jnp.float32)
        o2[...] = jnp.dot(s, w2_nr[...] @ l2_nr[...],
                          preferred_element_type=jnp.float32)

    full = lambda a: pl.BlockSpec(a.shape, lambda i: tuple(0 for _ in a.shape))
    o = jax.ShapeDtypeStruct((N, Fo), jnp.float32)
    return pl.pallas_call(
        body,
        grid=(N // _BN,),
        in_specs=[pl.BlockSpec((_BN, F), lambda i: (i, 0)),
                  pl.BlockSpec((2, _NC, _BN, F), lambda i: (0, 0, i, 0)),
                  full(bl), full(b1), full(b2), full(lnb),
                  full(l0), full(l1), full(l2),
                  full(wl_n), full(l0_n), full(w1_n), full(l1_n),
                  full(w2_n), full(l2_n)],
        out_specs=[pl.BlockSpec((_BN, Fo), lambda i: (i, 0))] * 3,
        out_shape=[o, o, o],
    )(g0, seg, bl, b1, b2, lnb, l0, l1, l2, wl_n, l0_n, w1_n, l1_n, w2_n, l2_n)


def _final_stage(g0, seg, bl, b1, b2, lnb, l0, l1, l2, C):
    N, F = g0.shape

    def body(g0_r, seg_r, bl_r, b1_r, b2_r, lnb_r, l0_r, l1_r, l2_r, o):
        c = (bl_r[...] @ l0_r[...] + b1_r[...] @ l1_r[...]
             + b2_r[...] @ l2_r[...] + lnb_r[...])
        s = (g0_r[...] + seg_r[0, 0] + seg_r[0, 1] + seg_r[1, 0]
             + seg_r[1, 1])
        z = s[:, :C] + c
        m = jnp.max(z, axis=1, keepdims=True)
        e = jnp.exp(z - m)
        lse = jnp.log(jnp.sum(e, axis=1, keepdims=True)) + m
        o[...] = z - lse

    full = lambda a: pl.BlockSpec(a.shape, lambda i: tuple(0 for _ in a.shape))
    return pl.pallas_call(
        body,
        grid=(N // _BN,),
        in_specs=[pl.BlockSpec((_BN, F), lambda i: (i, 0)),
                  pl.BlockSpec((2, _NC, _BN, F), lambda i: (0, 0, i, 0)),
                  full(bl), full(b1), full(b2), full(lnb),
                  full(l0), full(l1), full(l2)],
        out_specs=pl.BlockSpec((_BN, C), lambda i: (i, 0)),
        out_shape=jax.ShapeDtypeStruct((N, C), jnp.float32),
    )(g0, seg, bl, b1, b2, lnb, l0, l1, l2)


def kernel(x, edge_index, edge_weight, edge_index2, edge_weight2,
           ib1_ln_W, ib1_ln_b, ib1_c1_W, ib1_c1_b, ib1_c2_W, ib1_c2_b,
           ln1_W, ln1_b,
           ib2_ln_W, ib2_ln_b, ib2_c1_W, ib2_c1_b, ib2_c2_W, ib2_c2_b,
           ln2_W, ln2_b,
           ib3_ln_W, ib3_ln_b, ib3_c1_W, ib3_c1_b, ib3_c2_W, ib3_c2_b,
           ln3_W, ln3_b):
    N, _ = x.shape
    E = edge_weight.shape[0]
    Hd = ib1_ln_W.shape[1]
    C = ln3_W.shape[1]
    Cp = 48

    K = 80
    NCH = E // _NW // K
    esh = lambda v: v.reshape(_NW, NCH, K)
    src1, dst1 = esh(edge_index[0]), esh(edge_index[1])
    src2, dst2 = esh(edge_index2[0]), esh(edge_index2[1])
    ew1, ew2 = esh(edge_weight), esh(edge_weight2)

    l1 = [ln1_W[i * Hd:(i + 1) * Hd] for i in range(3)]
    l2 = [ln2_W[i * Hd:(i + 1) * Hd] for i in range(3)]
    l3 = [ln3_W[i * C:(i + 1) * C] for i in range(3)]
    l3p = [jnp.pad(m, ((0, 0), (0, Cp - C))) for m in l3]
    row = lambda v: v.reshape(1, -1)

    g0, g1, g2 = _first_proj(x, ib1_ln_W, l1[0], ib1_c1_W, l1[1],
                             ib1_c2_W, l1[2])
    seg = _seg_pair(N, Hd, E)(g1, g2, src1, dst1, ew1, src2, dst2, ew2)

    g0, g1, g2 = _combine_proj(g0, seg, row(ib1_ln_b), row(ib1_c1_b),
                               row(ib1_c2_b), row(ln1_b), l1[0], l1[1], l1[2],
                               ib2_ln_W, l2[0], ib2_c1_W, l2[1],
                               ib2_c2_W, l2[2])
    seg = _seg_pair(N, Hd, E)(g1, g2, src1, dst1, ew1, src2, dst2, ew2)

    g0, g1, g2 = _combine_proj(g0, seg, row(ib2_ln_b), row(ib2_c1_b),
                               row(ib2_c2_b), row(ln2_b), l2[0], l2[1], l2[2],
                               ib3_ln_W, l3p[0], ib3_c1_W, l3p[1],
                               ib3_c2_W, l3p[2])
    seg = _seg_pair(N, Cp, E)(g1, g2, src1, dst1, ew1, src2, dst2, ew2)

    return _final_stage(g0, seg, row(ib3_ln_b), row(ib3_c1_b),
                        row(ib3_c2_b), row(ln3_b), l3[0], l3[1], l3[2], C)


# speedup vs baseline: 13.2256x; 1.1127x over previous
import functools

import jax
import jax.numpy as jnp
from jax import lax
from jax.experimental import pallas as pl
from jax.experimental.pallas import tpu as pltpu
from jax.experimental.pallas import tpu_sc as plsc

_NC, _NS = 2, 16
_NW = _NC * _NS


@functools.partial(functools.cache)
def _seg_pair(N, F, E):
    K = 80
    EPW = E // _NW
    NCH = EPW // K
    RPT = (N // _NS) // 8 * 8
    TAIL = N - _NS * RPT
    ZR = 48
    NZ = RPT // ZR
    assert EPW % K == 0 and RPT % ZR == 0 and F % 16 == 0
    assert TAIL % 8 == 0 and TAIL <= ZR

    mesh = plsc.VectorSubcoreMesh(core_axis_name="c", subcore_axis_name="s",
                                  num_cores=_NC, num_subcores=_NS)

    def body(g1, g2, ei1, ew1, ei2, ew2, out,
             table, acc, zbuf, src_flat, dst_flat, ew_flat,
             rows_a, rows_b, rows_c, gsem0, gsem1, gsem2,
             ssem0, ssem1, ssem2):
        cid = lax.axis_index("c")
        sid = lax.axis_index("s")
        r0 = sid * RPT
        z16 = jnp.zeros((16,), jnp.float32)

        def zrow(i, carry):
            for j in range(F // 16):
                zbuf[i, pl.ds(j * 16, 16)] = z16
            return carry
        lax.fori_loop(0, ZR, zrow, 0)

        def stage(g_hbm, zero):
            pltpu.sync_copy(g_hbm.at[pl.ds(r0, RPT)], table.at[pl.ds(r0, RPT)])
            if zero:
                for i in range(NZ):
                    pltpu.sync_copy(zbuf, acc.at[pl.ds(r0 + i * ZR, ZR)])
            if TAIL:
                @pl.when(sid == _NS - 1)
                def _():
                    t0 = _NS * RPT
                    pltpu.sync_copy(g_hbm.at[pl.ds(t0, TAIL)],
                                    table.at[pl.ds(t0, TAIL)])
                    if zero:
                        pltpu.sync_copy(zbuf.at[pl.ds(0, TAIL)],
                                        acc.at[pl.ds(t0, TAIL)])

        def multiply(rows, ci):
            for kb in range(K // 16):
                off = pl.multiple_of(ci * K + kb * 16, 16)
                wv = ew_flat[pl.ds(off, 16)]
                for t in range(16):
                    k = kb * 16 + t
                    w = wv[t]
                    for j in range(F // 16):
                        sl = pl.ds(j * 16, 16)
                        rows[k, sl] = rows[k, sl] * w

        def edge_loop(ei_hbm, ew_hbm):
            wid = cid * _NS + sid
            e0 = pl.multiple_of(wid * EPW, 8)
            pltpu.sync_copy(ei_hbm.at[0, pl.ds(e0, EPW)], src_flat)
            pltpu.sync_copy(ei_hbm.at[1, pl.ds(e0, EPW)], dst_flat)
            pltpu.sync_copy(ew_hbm.at[pl.ds(e0, EPW)], ew_flat)

            rows = [rows_a, rows_b, rows_c]
            gsem = [gsem0, gsem1, gsem2]
            ssem = [ssem0, ssem1, ssem2]

            def handle(i, u):
                @pl.when((i >= 3) & (i < NCH))
                def _():
                    pltpu.make_async_copy(
                        rows[u], acc.at[dst_flat.at[pl.ds(pl.multiple_of((i - 3) * K, 8), K)]], ssem[u],
                    ).wait()

                @pl.when(i < NCH)
                def _():
                    pltpu.async_copy(table.at[src_flat.at[pl.ds(pl.multiple_of(i * K, 8), K)]], rows[u],
                                     gsem[u])

                v = (u - 1) % 3

                @pl.when((i >= 1) & (i - 1 < NCH))
                def _():
                    pltpu.make_async_copy(table.at[src_flat.at[pl.ds(pl.multiple_of((i - 1) * K, 8), K)]],
                                          rows[v], gsem[v]).wait()
                    multiply(rows[v], i - 1)
                    pltpu.async_copy(rows[v], acc.at[dst_flat.at[pl.ds(pl.multiple_of((i - 1) * K, 8), K)]],
                                     ssem[v], add=True)

            NIT = (NCH + 1 + 2) // 3

            def step(t, carry):
                for u in range(3):
                    handle(3 * t + u, u)
                return carry
            lax.fori_loop(0, NIT, step, 0)

            for c in range(max(0, NCH - 3), NCH):
                pltpu.make_async_copy(rows[c % 3], acc.at[dst_flat.at[pl.ds(c * K, K)]],
                                      ssem[c % 3]).wait()

        def writeback():
            pltpu.sync_copy(acc.at[pl.ds(r0, RPT)],
                            out.at[cid, pl.ds(r0, RPT)])
            if TAIL:
                @pl.when(sid == _NS - 1)
                def _():
                    t0 = _NS * RPT
                    pltpu.sync_copy(acc.at[pl.ds(t0, TAIL)],
                                    out.at[cid, pl.ds(t0, TAIL)])

        stage(g1, zero=True)
        plsc.subcore_barrier()
        edge_loop(ei1, ew1)
        plsc.subcore_barrier()
        stage(g2, zero=False)
        plsc.subcore_barrier()
        edge_loop(ei2, ew2)
        plsc.subcore_barrier()
        writeback()

    return pl.kernel(
        body,
        out_type=jax.ShapeDtypeStruct((_NC, N, F), jnp.float32),
        mesh=mesh,
        compiler_params=pltpu.CompilerParams(use_tc_tiling_on_sc=False),
        scratch_types=[
            pltpu.VMEM_SHARED((N, F), jnp.float32),
            pltpu.VMEM_SHARED((N, F), jnp.float32),
            pltpu.VMEM((ZR, F), jnp.float32),
            pltpu.VMEM((EPW,), jnp.int32),
            pltpu.VMEM((EPW,), jnp.int32),
            pltpu.VMEM((EPW,), jnp.float32),
            pltpu.VMEM((K, F), jnp.float32),
            pltpu.VMEM((K, F), jnp.float32),
            pltpu.VMEM((K, F), jnp.float32),
            pltpu.SemaphoreType.DMA,
            pltpu.SemaphoreType.DMA,
            pltpu.SemaphoreType.DMA,
            pltpu.SemaphoreType.DMA,
            pltpu.SemaphoreType.DMA,
            pltpu.SemaphoreType.DMA,
        ],
    )


_BN = 2000


def _first_proj(x, wl, l0, w1, l1, w2, l2):
    N, Din = x.shape
    F = l0.shape[1]

    def body(x_ref, wl_r, l0_r, w1_r, l1_r, w2_r, l2_r, o0, o1, o2):
        xx = x_ref[...]
        o0[...] = jnp.dot(xx, wl_r[...] @ l0_r[...],
                          preferred_element_type=jnp.float32)
        o1[...] = jnp.dot(xx, w1_r[...] @ l1_r[...],
                          preferred_element_type=jnp.float32)
        o2[...] = jnp.dot(xx, w2_r[...] @ l2_r[...],
                          preferred_element_type=jnp.float32)

    full = lambda a: pl.BlockSpec(a.shape, lambda i: (0, 0))
    o = jax.ShapeDtypeStruct((N, F), jnp.float32)
    return pl.pallas_call(
        body,
        grid=(N // _BN,),
        in_specs=[pl.BlockSpec((_BN, Din), lambda i: (i, 0)),
                  full(wl), full(l0), full(w1), full(l1), full(w2), full(l2)],
        out_specs=[pl.BlockSpec((_BN, F), lambda i: (i, 0))] * 3,
        out_shape=[o, o, o],
    )(x, wl, l0, w1, l1, w2, l2)


def _combine_proj(g0, seg, bl, b1, b2, lnb, l0, l1, l2, wl_n, l0_n, w1_n, l1_n,
                  w2_n, l2_n):
    N, F = g0.shape
    Fo = l0_n.shape[1]

    def body(g0_r, seg_r, bl_r, b1_r, b2_r, lnb_r, l0_r, l1_r, l2_r,
             wl_nr, l0_nr, w1_nr, l1_nr, w2_nr, l2_nr, o0, o1, o2):
        c = (bl_r[...] @ l0_r[...] + b1_r[...] @ l1_r[...]
             + b2_r[...] @ l2_r[...] + lnb_r[...])
        s = g0_r[...] + seg_r[0] + seg_r[1] + c
        o0[...] = jnp.dot(s, wl_nr[...] @ l0_nr[...],
                          preferred_element_type=jnp.float32)
        o1[...] = jnp.dot(s, w1_nr[...] @ l1_nr[...],
                          preferred_element_type=jnp.float32)
        o2[...] = jnp.dot(s, w2_nr[...] @ l2_nr[...],
                          preferred_element_type=jnp.float32)

    full = lambda a: pl.BlockSpec(a.shape, lambda i: tuple(0 for _ in a.shape))
    o = jax.ShapeDtypeStruct((N, Fo), jnp.float32)
    return pl.pallas_call(
        body,
        grid=(N // _BN,),
        in_specs=[pl.BlockSpec((_BN, F), lambda i: (i, 0)),
                  pl.BlockSpec((_NC, _BN, F), lambda i: (0, i, 0)),
                  full(bl), full(b1), full(b2), full(lnb),
                  full(l0), full(l1), full(l2),
                  full(wl_n), full(l0_n), full(w1_n), full(l1_n),
                  full(w2_n), full(l2_n)],
        out_specs=[pl.BlockSpec((_BN, Fo), lambda i: (i, 0))] * 3,
        out_shape=[o, o, o],
    )(g0, seg, bl, b1, b2, lnb, l0, l1, l2, wl_n, l0_n, w1_n, l1_n, w2_n, l2_n)


def _final_stage(g0, seg, bl, b1, b2, lnb, l0, l1, l2, C):
    N, F = g0.shape

    def body(g0_r, seg_r, bl_r, b1_r, b2_r, lnb_r, l0_r, l1_r, l2_r, o):
        c = (bl_r[...] @ l0_r[...] + b1_r[...] @ l1_r[...]
             + b2_r[...] @ l2_r[...] + lnb_r[...])
        s = g0_r[...] + seg_r[0] + seg_r[1]
        z = s[:, :C] + c
        m = jnp.max(z, axis=1, keepdims=True)
        e = jnp.exp(z - m)
        lse = jnp.log(jnp.sum(e, axis=1, keepdims=True)) + m
        o[...] = z - lse

    full = lambda a: pl.BlockSpec(a.shape, lambda i: tuple(0 for _ in a.shape))
    return pl.pallas_call(
        body,
        grid=(N // _BN,),
        in_specs=[pl.BlockSpec((_BN, F), lambda i: (i, 0)),
                  pl.BlockSpec((_NC, _BN, F), lambda i: (0, i, 0)),
                  full(bl), full(b1), full(b2), full(lnb),
                  full(l0), full(l1), full(l2)],
        out_specs=pl.BlockSpec((_BN, C), lambda i: (i, 0)),
        out_shape=jax.ShapeDtypeStruct((N, C), jnp.float32),
    )(g0, seg, bl, b1, b2, lnb, l0, l1, l2)


def kernel(x, edge_index, edge_weight, edge_index2, edge_weight2,
           ib1_ln_W, ib1_ln_b, ib1_c1_W, ib1_c1_b, ib1_c2_W, ib1_c2_b,
           ln1_W, ln1_b,
           ib2_ln_W, ib2_ln_b, ib2_c1_W, ib2_c1_b, ib2_c2_W, ib2_c2_b,
           ln2_W, ln2_b,
           ib3_ln_W, ib3_ln_b, ib3_c1_W, ib3_c1_b, ib3_c2_W, ib3_c2_b,
           ln3_W, ln3_b):
    N, _ = x.shape
    E = edge_weight.shape[0]
    Hd = ib1_ln_W.shape[1]
    C = ln3_W.shape[1]
    Cp = 48


    l1 = [ln1_W[i * Hd:(i + 1) * Hd] for i in range(3)]
    l2 = [ln2_W[i * Hd:(i + 1) * Hd] for i in range(3)]
    l3 = [ln3_W[i * C:(i + 1) * C] for i in range(3)]
    l3p = [jnp.pad(m, ((0, 0), (0, Cp - C))) for m in l3]
    row = lambda v: v.reshape(1, -1)

    g0, g1, g2 = _first_proj(x, ib1_ln_W, l1[0], ib1_c1_W, l1[1],
                             ib1_c2_W, l1[2])
    seg = _seg_pair(N, Hd, E)(g1, g2, edge_index, edge_weight,
                              edge_index2, edge_weight2)

    g0, g1, g2 = _combine_proj(g0, seg, row(ib1_ln_b), row(ib1_c1_b),
                               row(ib1_c2_b), row(ln1_b), l1[0], l1[1], l1[2],
                               ib2_ln_W, l2[0], ib2_c1_W, l2[1],
                               ib2_c2_W, l2[2])
    seg = _seg_pair(N, Hd, E)(g1, g2, edge_index, edge_weight,
                              edge_index2, edge_weight2)

    g0, g1, g2 = _combine_proj(g0, seg, row(ib2_ln_b), row(ib2_c1_b),
                               row(ib2_c2_b), row(ln2_b), l2[0], l2[1], l2[2],
                               ib3_ln_W, l3p[0], ib3_c1_W, l3p[1],
                               ib3_c2_W, l3p[2])
    seg = _seg_pair(N, Cp, E)(g1, g2, edge_index, edge_weight,
                              edge_index2, edge_weight2)

    return _final_stage(g0, seg, row(ib3_ln_b), row(ib3_c1_b),
                        row(ib3_c2_b), row(ln3_b), l3[0], l3[1], l3[2], C)
